# baseline (speedup 1.0000x reference)
import functools
import jax
import jax.numpy as jnp
from jax import lax
from jax.experimental import pallas as pl
from jax.experimental.pallas import tpu as pltpu
from jax.experimental.pallas import tpu_sc as plsc

_ROWS = 4096 * 26
_CARD = 1000
_NC = 2
_NS = 16
_NW = _NC * _NS
_RPW = _ROWS // _NW
_CHUNK = 32
_NCHUNK = _RPW // _CHUNK
_BUF = _CHUNK * _CARD


@functools.partial(
    pl.kernel,
    mesh=plsc.VectorSubcoreMesh(core_axis_name="c", subcore_axis_name="s"),
    out_type=jax.ShapeDtypeStruct((_ROWS * _CARD,), jnp.float32),
    scratch_types=[
        pltpu.VMEM((_RPW,), jnp.int32),
        pltpu.VMEM((_BUF,), jnp.float32),
    ],
    compiler_params=pltpu.CompilerParams(use_tc_tiling_on_sc=False, needs_layout_passes=False),
)
def _sc_onehot(xe_hbm, out_hbm, idx_v, buf):
    wid = lax.axis_index("s") * _NC + lax.axis_index("c")
    base_row = wid * _RPW
    pltpu.sync_copy(xe_hbm.at[pl.ds(base_row, _RPW)], idx_v)

    zeros16 = jnp.zeros((16,), jnp.float32)
    ones16 = jnp.ones((16,), jnp.float32)
    riota = lax.iota(jnp.int32, 16)

    def zbody(i, carry):
        buf[pl.ds(pl.multiple_of(i * 16, 16), 16)] = zeros16
        return carry

    lax.fori_loop(0, _BUF // 16, zbody, 0)

    def chunk_body(c, carry):
        for g in range(_CHUNK // 16):
            vals = idx_v[pl.ds(c * _CHUNK + g * 16, 16)]
            flat = (g * 16 + riota) * _CARD + vals
            plsc.store_scatter(buf, [flat], ones16)
        pltpu.sync_copy(
            buf, out_hbm.at[pl.ds((base_row + c * _CHUNK) * _CARD, _BUF)]
        )
        for g in range(_CHUNK // 16):
            vals = idx_v[pl.ds(c * _CHUNK + g * 16, 16)]
            flat = (g * 16 + riota) * _CARD + vals
            plsc.store_scatter(buf, [flat], zeros16)
        return carry

    lax.fori_loop(0, _NCHUNK, chunk_body, 0)


def kernel(xe):
    out = _sc_onehot(xe.reshape(-1))
    return out.reshape(4096, 26000)


# speedup vs baseline: 2.0074x; 2.0074x over previous
import numpy as np
import jax
import jax.numpy as jnp
from jax.experimental import pallas as pl
from jax.experimental.pallas import tpu as pltpu

_NUM_FIELDS = 26
_CARD = 1000
_OUT_COLS = _NUM_FIELDS * _CARD
_COL_BLOCK = 2048
_NCB = 13
_BR = 64
_NBUF = 4

_FIELD_IDX = np.minimum(
    (np.arange(_NCB) * _COL_BLOCK // _CARD)[:, None] + np.arange(3)[None, :],
    _NUM_FIELDS - 1,
)


def _body(tgt_ref, out_ref, buf, sems):
    i = pl.program_id(0)
    ni = pl.num_programs(0)
    s = lax.rem(i, _NBUF)

    @pl.when(i >= _NBUF)
    def _wait_prev():
        r = i - _NBUF
        pltpu.make_async_copy(
            buf.at[s],
            out_ref.at[pl.ds(r * _BR, _BR), :],
            sems.at[s],
        ).wait()

    iota = jax.lax.broadcasted_iota(jnp.int32, (_BR, _COL_BLOCK), 1)
    for j in range(_NCB):
        cols = j * _COL_BLOCK + iota
        m = (
            (cols == tgt_ref[j, :, 0:1])
            | (cols == tgt_ref[j, :, 1:2])
            | (cols == tgt_ref[j, :, 2:3])
        )
        v = m.astype(jnp.float32)
        w = min(_COL_BLOCK, _OUT_COLS - j * _COL_BLOCK)
        buf[s, :, pl.ds(j * _COL_BLOCK, w)] = v[:, :w]

    pltpu.async_copy(
        buf.at[s],
        out_ref.at[pl.ds(i * _BR, _BR), :],
        sems.at[s],
    )

    @pl.when(i == ni - 1)
    def _drain():
        for k in range(_NBUF):
            r = i - k
            @pl.when(r >= 0)
            def _():
                pltpu.make_async_copy(
                    buf.at[lax.rem(r, _NBUF)],
                    out_ref.at[pl.ds(r * _BR, _BR), :],
                    sems.at[lax.rem(r, _NBUF)],
                ).wait()


from jax import lax


def kernel(xe):
    b = xe.shape[0]
    fidx = jnp.asarray(_FIELD_IDX.reshape(-1), dtype=jnp.int32)
    tgt = jnp.take(xe, fidx, axis=1).astype(jnp.int32) + fidx * _CARD
    tgt = tgt.reshape(b, _NCB, 3).transpose(1, 0, 2)
    return pl.pallas_call(
        _body,
        grid=(b // _BR,),
        in_specs=[pl.BlockSpec((_NCB, _BR, 3), lambda r: (0, r, 0))],
        out_specs=pl.BlockSpec(memory_space=pltpu.MemorySpace.HBM),
        out_shape=jax.ShapeDtypeStruct((b, _OUT_COLS), jnp.float32),
        scratch_shapes=[
            pltpu.VMEM((_NBUF, _BR, _OUT_COLS), jnp.float32),
            pltpu.SemaphoreType.DMA((_NBUF,)),
        ],
    )(tgt)
